# Initial kernel scaffold; baseline (speedup 1.0000x reference)
import jax
import jax.numpy as jnp
from jax.experimental import pallas as pl


def kernel(pred_logits, pred_boxes, orig_target_sizes):
    raise NotImplementedError("write your pallas kernel here")


# speedup vs baseline: 1.3020x; 1.3020x over previous
import jax
import jax.numpy as jnp
from jax.experimental import pallas as pl
from jax.experimental.pallas import tpu as pltpu

_B, _Q, _C = 8, 5000, 80
_K = 300
_ROWS = 3200
_LANES = 128


def _postproc_kernel(logits_ref, boxes_ref, scale_ref,
                     labels_ref, boxes_out_ref, scores_ref,
                     x_s, rmax_s):
    neg = jnp.float32(-jnp.inf)
    x_s[...] = logits_ref[0]
    x3 = x_s[...].reshape(_ROWS // _LANES, _LANES, _LANES)
    rmax_s[...] = jnp.max(x3, axis=2)

    lane_iota = jax.lax.broadcasted_iota(jnp.int32, (1, _LANES), 1)
    row_iota = (jax.lax.broadcasted_iota(jnp.int32, (_ROWS // _LANES, _LANES), 0) * _LANES
                + jax.lax.broadcasted_iota(jnp.int32, (_ROWS // _LANES, _LANES), 1))
    big = jnp.int32(2 ** 30)

    def body(k, _):
        rm = rmax_s[...]
        v = jnp.max(rm)
        r = jnp.min(jnp.where(rm == v, row_iota, big))
        row = x_s[pl.ds(r, 1), :]
        l = jnp.min(jnp.where(row == v, lane_iota, big))
        flat = r * _LANES + l
        q = flat // _C
        c = flat - q * _C
        labels_ref[0, pl.ds(k, 1), :] = jnp.full((1, 1), c, jnp.int32)
        scores_ref[0, pl.ds(k, 1), :] = jnp.full((1, 1), v, jnp.float32)
        boxes_out_ref[0, pl.ds(k, 1), :] = boxes_ref[0, pl.ds(q, 1), :]
        row2 = jnp.where(lane_iota == l, neg, row)
        x_s[pl.ds(r, 1), :] = row2
        rmax_s[...] = jnp.where(row_iota == r, jnp.max(row2), rm)
        return 0

    jax.lax.fori_loop(0, _K, body, 0)

    scores_ref[0] = jax.nn.sigmoid(scores_ref[0])
    bx = boxes_out_ref[0]
    cx, cy, w, h = bx[:, 0:1], bx[:, 1:2], bx[:, 2:3], bx[:, 3:4]
    s = scale_ref[0]
    x0 = (cx - 0.5 * w) * s[:, 0:1]
    y0 = (cy - 0.5 * h) * s[:, 1:2]
    ww = w * s[:, 2:3]
    hh = h * s[:, 3:4]
    boxes_out_ref[0] = jnp.concatenate(
        [jnp.maximum(x0, 0.0), jnp.maximum(y0, 0.0),
         jnp.maximum(ww, 1.0), jnp.maximum(hh, 1.0)], axis=1)


def kernel(pred_logits, pred_boxes, orig_target_sizes):
    b, q, c = pred_logits.shape
    flat = pred_logits.reshape(b, q * c)
    pad = _ROWS * _LANES - q * c
    flat = jnp.pad(flat, ((0, 0), (0, pad)), constant_values=-jnp.inf)
    flat = flat.reshape(b, _ROWS, _LANES)

    sizes = orig_target_sizes.astype(jnp.float32)
    scale_wh = jnp.stack([sizes[:, 1], sizes[:, 0],
                          sizes[:, 1], sizes[:, 0]], axis=1)
    scale_wh = scale_wh[:, None, :]

    labels, boxes, scores = pl.pallas_call(
        _postproc_kernel,
        grid=(b,),
        in_specs=[
            pl.BlockSpec((1, _ROWS, _LANES), lambda i: (i, 0, 0)),
            pl.BlockSpec((1, q, 4), lambda i: (i, 0, 0)),
            pl.BlockSpec((1, 1, 4), lambda i: (i, 0, 0)),
        ],
        out_specs=[
            pl.BlockSpec((1, _K, 1), lambda i: (i, 0, 0)),
            pl.BlockSpec((1, _K, 4), lambda i: (i, 0, 0)),
            pl.BlockSpec((1, _K, 1), lambda i: (i, 0, 0)),
        ],
        out_shape=[
            jax.ShapeDtypeStruct((b, _K, 1), jnp.int32),
            jax.ShapeDtypeStruct((b, _K, 4), jnp.float32),
            jax.ShapeDtypeStruct((b, _K, 1), jnp.float32),
        ],
        scratch_shapes=[
            pltpu.VMEM((_ROWS, _LANES), jnp.float32),
            pltpu.VMEM((_ROWS // _LANES, _LANES), jnp.float32),
        ],
        compiler_params=pltpu.CompilerParams(
            dimension_semantics=("arbitrary",)),
    )(flat, pred_boxes, scale_wh)

    return labels[..., 0], boxes, scores[..., 0]


# speedup vs baseline: 1.3148x; 1.0098x over previous
import jax
import jax.numpy as jnp
from jax.experimental import pallas as pl
from jax.experimental.pallas import tpu as pltpu

_B, _Q, _C = 8, 5000, 80
_K = 300
_ROWS = 3200
_LANES = 128


def _postproc_kernel(logits_ref, boxes_ref, scale_ref,
                     labels_ref, boxes_out_ref, scores_ref,
                     x_s, rmax_s):
    neg = jnp.float32(-jnp.inf)
    x_s[...] = logits_ref[0]
    x3 = x_s[...].reshape(_ROWS // _LANES, _LANES, _LANES)
    rmax_s[...] = jnp.max(x3, axis=2)

    lane_iota = jax.lax.broadcasted_iota(jnp.int32, (1, _LANES), 1)
    row_iota = (jax.lax.broadcasted_iota(jnp.int32, (_ROWS // _LANES, _LANES), 0) * _LANES
                + jax.lax.broadcasted_iota(jnp.int32, (_ROWS // _LANES, _LANES), 1))
    big = jnp.int32(2 ** 30)

    def body(k, rm):
        v = jnp.max(rm)
        r = jnp.min(jnp.where(rm == v, row_iota, big))
        row = x_s[pl.ds(r, 1), :]
        l = jnp.min(jnp.where(row == v, lane_iota, big))
        flat = r * _LANES + l
        q = flat // _C
        c = flat - q * _C
        labels_ref[0, pl.ds(k, 1), :] = jnp.full((1, 1), c, jnp.int32)
        scores_ref[0, pl.ds(k, 1), :] = jnp.full((1, 1), v, jnp.float32)
        boxes_out_ref[0, pl.ds(k, 1), :] = boxes_ref[0, pl.ds(q, 1), :]
        row2 = jnp.where(lane_iota == l, neg, row)
        x_s[pl.ds(r, 1), :] = row2
        return jnp.where(row_iota == r, jnp.max(row2), rm)

    jax.lax.fori_loop(0, _K, body, rmax_s[...])

    scores_ref[0] = jax.nn.sigmoid(scores_ref[0])
    bx = boxes_out_ref[0]
    cx, cy, w, h = bx[:, 0:1], bx[:, 1:2], bx[:, 2:3], bx[:, 3:4]
    s = scale_ref[0]
    x0 = (cx - 0.5 * w) * s[:, 0:1]
    y0 = (cy - 0.5 * h) * s[:, 1:2]
    ww = w * s[:, 2:3]
    hh = h * s[:, 3:4]
    boxes_out_ref[0] = jnp.concatenate(
        [jnp.maximum(x0, 0.0), jnp.maximum(y0, 0.0),
         jnp.maximum(ww, 1.0), jnp.maximum(hh, 1.0)], axis=1)


def kernel(pred_logits, pred_boxes, orig_target_sizes):
    b, q, c = pred_logits.shape
    flat = pred_logits.reshape(b, q * c)
    pad = _ROWS * _LANES - q * c
    flat = jnp.pad(flat, ((0, 0), (0, pad)), constant_values=-jnp.inf)
    flat = flat.reshape(b, _ROWS, _LANES)

    sizes = orig_target_sizes.astype(jnp.float32)
    scale_wh = jnp.stack([sizes[:, 1], sizes[:, 0],
                          sizes[:, 1], sizes[:, 0]], axis=1)
    scale_wh = scale_wh[:, None, :]

    labels, boxes, scores = pl.pallas_call(
        _postproc_kernel,
        grid=(b,),
        in_specs=[
            pl.BlockSpec((1, _ROWS, _LANES), lambda i: (i, 0, 0)),
            pl.BlockSpec((1, q, 4), lambda i: (i, 0, 0)),
            pl.BlockSpec((1, 1, 4), lambda i: (i, 0, 0)),
        ],
        out_specs=[
            pl.BlockSpec((1, _K, 1), lambda i: (i, 0, 0)),
            pl.BlockSpec((1, _K, 4), lambda i: (i, 0, 0)),
            pl.BlockSpec((1, _K, 1), lambda i: (i, 0, 0)),
        ],
        out_shape=[
            jax.ShapeDtypeStruct((b, _K, 1), jnp.int32),
            jax.ShapeDtypeStruct((b, _K, 4), jnp.float32),
            jax.ShapeDtypeStruct((b, _K, 1), jnp.float32),
        ],
        scratch_shapes=[
            pltpu.VMEM((_ROWS, _LANES), jnp.float32),
            pltpu.VMEM((_ROWS // _LANES, _LANES), jnp.float32),
        ],
        compiler_params=pltpu.CompilerParams(
            dimension_semantics=("parallel",)),
    )(flat, pred_boxes, scale_wh)

    return labels[..., 0], boxes, scores[..., 0]


# speedup vs baseline: 1.5345x; 1.1671x over previous
import jax
import jax.numpy as jnp
from jax.experimental import pallas as pl
from jax.experimental.pallas import tpu as pltpu

_B, _Q, _C = 8, 5000, 80
_K = 300
_ROWS = 3200
_LANES = 128


def _postproc_kernel(logits_ref, boxes_ref, scale_ref,
                     labels_ref, boxes_out_ref, scores_ref, x_s):
    neg = jnp.float32(-jnp.inf)
    x_s[...] = logits_ref[...]
    x4 = x_s[...].reshape(_B, _ROWS // _LANES, _LANES, _LANES)
    rm0 = jnp.max(x4, axis=3)

    lane_iota = jax.lax.broadcasted_iota(jnp.int32, (1, _LANES), 1)
    row_iota = (jax.lax.broadcasted_iota(jnp.int32, (_ROWS // _LANES, _LANES), 0) * _LANES
                + jax.lax.broadcasted_iota(jnp.int32, (_ROWS // _LANES, _LANES), 1))
    big = jnp.int32(2 ** 30)

    def body(k, rms):
        new = []
        for b in range(_B):
            rm = rms[b]
            v = jnp.max(rm)
            r = jnp.min(jnp.where(rm == v, row_iota, big))
            row = x_s[b, pl.ds(r, 1), :]
            l = jnp.min(jnp.where(row == v, lane_iota, big))
            flat = r * _LANES + l
            q = flat // _C
            c = flat - q * _C
            labels_ref[b, pl.ds(k, 1), :] = jnp.full((1, 1), c, jnp.int32)
            scores_ref[b, pl.ds(k, 1), :] = jnp.full((1, 1), v, jnp.float32)
            boxes_out_ref[b, pl.ds(k, 1), :] = boxes_ref[b, pl.ds(q, 1), :]
            row2 = jnp.where(lane_iota == l, neg, row)
            x_s[b, pl.ds(r, 1), :] = row2
            new.append(jnp.where(row_iota == r, jnp.max(row2), rm))
        return tuple(new)

    jax.lax.fori_loop(0, _K, body, tuple(rm0[b] for b in range(_B)))

    scores_ref[...] = jax.nn.sigmoid(scores_ref[...])
    bx = boxes_out_ref[...]
    cx, cy, w, h = bx[..., 0:1], bx[..., 1:2], bx[..., 2:3], bx[..., 3:4]
    s = scale_ref[...]
    x0 = (cx - 0.5 * w) * s[..., 0:1]
    y0 = (cy - 0.5 * h) * s[..., 1:2]
    ww = w * s[..., 2:3]
    hh = h * s[..., 3:4]
    boxes_out_ref[...] = jnp.concatenate(
        [jnp.maximum(x0, 0.0), jnp.maximum(y0, 0.0),
         jnp.maximum(ww, 1.0), jnp.maximum(hh, 1.0)], axis=2)


def kernel(pred_logits, pred_boxes, orig_target_sizes):
    b, q, c = pred_logits.shape
    flat = pred_logits.reshape(b, q * c)
    pad = _ROWS * _LANES - q * c
    flat = jnp.pad(flat, ((0, 0), (0, pad)), constant_values=-jnp.inf)
    flat = flat.reshape(b, _ROWS, _LANES)

    sizes = orig_target_sizes.astype(jnp.float32)
    scale_wh = jnp.stack([sizes[:, 1], sizes[:, 0],
                          sizes[:, 1], sizes[:, 0]], axis=1)
    scale_wh = scale_wh[:, None, :]

    labels, boxes, scores = pl.pallas_call(
        _postproc_kernel,
        out_shape=[
            jax.ShapeDtypeStruct((b, _K, 1), jnp.int32),
            jax.ShapeDtypeStruct((b, _K, 4), jnp.float32),
            jax.ShapeDtypeStruct((b, _K, 1), jnp.float32),
        ],
        scratch_shapes=[
            pltpu.VMEM((b, _ROWS, _LANES), jnp.float32),
        ],
    )(flat, pred_boxes, scale_wh)

    return labels[..., 0], boxes, scores[..., 0]


# speedup vs baseline: 2.9869x; 1.9465x over previous
import jax
import jax.numpy as jnp
from jax.experimental import pallas as pl
from jax.experimental.pallas import tpu as pltpu

_B, _Q, _C = 8, 5000, 80
_K = 300
_ROWS = 3200
_LANES = 128


def _postproc_kernel(logits_ref, boxes_ref, scale_ref,
                     labels_ref, boxes_out_ref, scores_ref, x_s):
    neg = jnp.float32(-jnp.inf)
    x_s[...] = logits_ref[...]
    x4 = x_s[...].reshape(_B, _ROWS // _LANES, _LANES, _LANES)
    rm0 = jnp.max(x4, axis=3)

    lane_iota = jax.lax.broadcasted_iota(jnp.int32, (1, _LANES), 1)
    row_iota = (jax.lax.broadcasted_iota(jnp.int32, (_ROWS // _LANES, _LANES), 0) * _LANES
                + jax.lax.broadcasted_iota(jnp.int32, (_ROWS // _LANES, _LANES), 1))
    big = jnp.int32(2 ** 30)

    def body(k, rms):
        new = []
        for b in range(_B):
            rm = rms[b]
            v = jnp.max(jnp.max(rm, axis=0, keepdims=True),
                        axis=1, keepdims=True)
            eq = rm == v
            cand = jnp.where(eq, row_iota, big)
            r_s = jnp.min(cand)
            r_v = jnp.min(jnp.min(cand, axis=0, keepdims=True),
                          axis=1, keepdims=True)
            row = x_s[b, pl.ds(r_s, 1), :]
            lcand = jnp.where(row == v, lane_iota, big)
            l_s = jnp.min(lcand)
            l_v = jnp.min(lcand, axis=1, keepdims=True)
            flat_v = r_v * _LANES + l_v
            q_v = flat_v // _C
            labels_ref[b, pl.ds(k, 1), :] = flat_v - q_v * _C
            scores_ref[b, pl.ds(k, 1), :] = v
            q_s = (r_s * _LANES + l_s) // _C
            boxes_out_ref[b, pl.ds(k, 1), :] = boxes_ref[b, pl.ds(q_s, 1), :]
            row2 = jnp.where(lane_iota == l_v, neg, row)
            x_s[b, pl.ds(r_s, 1), :] = row2
            nv = jnp.max(row2, axis=1, keepdims=True)
            new.append(jnp.where(row_iota == r_v, nv, rm))
        return tuple(new)

    jax.lax.fori_loop(0, _K, body, tuple(rm0[b] for b in range(_B)))

    scores_ref[...] = jax.nn.sigmoid(scores_ref[...])
    bx = boxes_out_ref[...]
    cx, cy, w, h = bx[..., 0:1], bx[..., 1:2], bx[..., 2:3], bx[..., 3:4]
    s = scale_ref[...]
    x0 = (cx - 0.5 * w) * s[..., 0:1]
    y0 = (cy - 0.5 * h) * s[..., 1:2]
    ww = w * s[..., 2:3]
    hh = h * s[..., 3:4]
    boxes_out_ref[...] = jnp.concatenate(
        [jnp.maximum(x0, 0.0), jnp.maximum(y0, 0.0),
         jnp.maximum(ww, 1.0), jnp.maximum(hh, 1.0)], axis=2)


def kernel(pred_logits, pred_boxes, orig_target_sizes):
    b, q, c = pred_logits.shape
    flat = pred_logits.reshape(b, q * c)
    pad = _ROWS * _LANES - q * c
    flat = jnp.pad(flat, ((0, 0), (0, pad)), constant_values=-jnp.inf)
    flat = flat.reshape(b, _ROWS, _LANES)

    sizes = orig_target_sizes.astype(jnp.float32)
    scale_wh = jnp.stack([sizes[:, 1], sizes[:, 0],
                          sizes[:, 1], sizes[:, 0]], axis=1)
    scale_wh = scale_wh[:, None, :]

    labels, boxes, scores = pl.pallas_call(
        _postproc_kernel,
        out_shape=[
            jax.ShapeDtypeStruct((b, _K, 1), jnp.int32),
            jax.ShapeDtypeStruct((b, _K, 4), jnp.float32),
            jax.ShapeDtypeStruct((b, _K, 1), jnp.float32),
        ],
        scratch_shapes=[
            pltpu.VMEM((b, _ROWS, _LANES), jnp.float32),
        ],
    )(flat, pred_boxes, scale_wh)

    return labels[..., 0], boxes, scores[..., 0]
